# Initial kernel scaffold; baseline (speedup 1.0000x reference)
import jax
import jax.numpy as jnp
from jax.experimental import pallas as pl


def kernel(x, edge_index, Wk, bk, Wq, bq, Wv, bv, Ws, bias):
    raise NotImplementedError("write your pallas kernel here")


# speedup vs baseline: 5.8899x; 5.8899x over previous
import functools

import jax
import jax.numpy as jnp
from jax import lax
from jax.experimental import pallas as pl
from jax.experimental.pallas import tpu as pltpu
from jax.experimental.pallas import tpu_sc as plsc

N = 10000
NPAD = 10240
F = 128
E = 160000
BT = 2
NSUB = 16
HALF = NPAD // 2
TRASH = HALF
ROWS_PER_TILE = HALF // NSUB
EDGES_PER_TILE = E // NSUB
CHUNK = 80
NCHUNK = EDGES_PER_TILE // CHUNK
LANES = 16


def _linear_body(x_ref, wk_ref, bk_ref, wqv_ref, bqv_ref, ws_ref, bs_ref,
                 k_ref, qv_ref, skip_ref):
  xb = x_ref[...]
  k_ref[...] = (
      jnp.dot(xb, wk_ref[...], preferred_element_type=jnp.float32)
      + bk_ref[...])
  qv_ref[...] = (
      jnp.dot(xb, wqv_ref[...], preferred_element_type=jnp.float32)
      + bqv_ref[...])
  skip_ref[...] = (
      jnp.dot(xb, ws_ref[...], preferred_element_type=jnp.float32)
      + bs_ref[...])


def _tc_linear(x_flat, wk, bk, wqv, bqv, ws, bs):
  rows = x_flat.shape[0]
  blk = 640
  grid = rows // blk
  full = lambda i: (0, 0)
  out = pl.pallas_call(
      _linear_body,
      grid=(grid,),
      in_specs=[
          pl.BlockSpec((blk, F), lambda i: (i, 0)),
          pl.BlockSpec((F, F), full),
          pl.BlockSpec((1, F), full),
          pl.BlockSpec((F, 2 * F), full),
          pl.BlockSpec((1, 2 * F), full),
          pl.BlockSpec((F, F), full),
          pl.BlockSpec((1, F), full),
      ],
      out_specs=[
          pl.BlockSpec((blk, F), lambda i: (i, 0)),
          pl.BlockSpec((blk, 2 * F), lambda i: (i, 0)),
          pl.BlockSpec((blk, F), lambda i: (i, 0)),
      ],
      out_shape=[
          jax.ShapeDtypeStruct((rows, F), jnp.float32),
          jax.ShapeDtypeStruct((rows, 2 * F), jnp.float32),
          jax.ShapeDtypeStruct((rows, F), jnp.float32),
      ],
  )(x_flat, wk, bk, wqv, bqv, ws, bs)
  return out


def _sc_body(k_hbm, qv_hbm, skip_hbm, src_hbm, dst_hbm, out_hbm,
             agg_sh, slab_v, kbuf, qvbuf, sidx, didx, sidx_off, didx_off,
             scat_idx, sem_k, sem_qv):
  c = lax.axis_index("c")
  t = lax.axis_index("s")
  cH = jnp.full((LANES,), c * HALF, dtype=jnp.int32)
  trash = jnp.full((LANES,), TRASH, dtype=jnp.int32)

  for b in range(BT):
    row0 = b * NPAD + c * HALF + t * ROWS_PER_TILE

    pltpu.sync_copy(skip_hbm.at[pl.ds(row0, ROWS_PER_TILE)], slab_v)
    pltpu.sync_copy(slab_v,
                    agg_sh.at[pl.ds(t * ROWS_PER_TILE, ROWS_PER_TILE)])
    plsc.subcore_barrier()

    bN = jnp.full((LANES,), b * NPAD, dtype=jnp.int32)

    def chunk_body(ci, carry):
      base = t * EDGES_PER_TILE + ci * CHUNK
      pltpu.sync_copy(src_hbm.at[pl.ds(base, CHUNK)], sidx)
      pltpu.sync_copy(dst_hbm.at[pl.ds(base, CHUNK)], didx)
      for i in range(CHUNK // LANES):
        sl = pl.ds(i * LANES, LANES)
        s = sidx[sl]
        d = didx[sl]
        sidx_off[sl] = s + bN
        didx_off[sl] = d + bN
        ld = d - cH
        keep = (d >= cH) & (ld < HALF)
        scat_idx[sl] = jnp.where(keep, ld, trash)
      cp_k = pltpu.async_copy(k_hbm.at[didx_off], kbuf, sem_k)
      cp_qv = pltpu.async_copy(qv_hbm.at[sidx_off], qvbuf, sem_qv)
      cp_k.wait()
      cp_qv.wait()

      def edge_body(e, c2):
        for j in range(F // LANES):
          sl = pl.ds(j * LANES, LANES)
          eta = kbuf[e, sl] + qvbuf[e, sl]
          eta = jnp.maximum(eta, eta * 0.01)
          kbuf[e, sl] = eta * qvbuf[e, pl.ds(F + j * LANES, LANES)]
        return c2

      lax.fori_loop(0, CHUNK, edge_body, 0)
      pltpu.sync_copy(kbuf, agg_sh.at[scat_idx], add=True)
      return carry

    lax.fori_loop(0, NCHUNK, chunk_body, 0)
    plsc.subcore_barrier()

    pltpu.sync_copy(agg_sh.at[pl.ds(t * ROWS_PER_TILE, ROWS_PER_TILE)],
                    slab_v)
    pltpu.sync_copy(slab_v, out_hbm.at[pl.ds(row0, ROWS_PER_TILE)])
    plsc.subcore_barrier()


def kernel(x, edge_index, Wk, bk, Wq, bq, Wv, bv, Ws, bias):
  b, t, n, f = x.shape
  x_pad = jnp.pad(x.reshape(b * t, n, f), ((0, 0), (0, NPAD - n), (0, 0)))
  x_flat = x_pad.reshape(b * t * NPAD, f)
  wqv = jnp.concatenate([Wq, Wv], axis=1)
  bqv = jnp.concatenate([bq, bv])[None, :]
  k, qv, skip = _tc_linear(x_flat, Wk, bk[None, :], wqv, bqv, Ws,
                           bias[None, :])

  src = edge_index[0]
  dst = edge_index[1]

  mesh = plsc.VectorSubcoreMesh(core_axis_name="c", subcore_axis_name="s")
  out = pl.kernel(
      _sc_body,
      out_type=jax.ShapeDtypeStruct((b * t * NPAD, f), jnp.float32),
      mesh=mesh,
      compiler_params=pltpu.CompilerParams(use_tc_tiling_on_sc=False),
      scratch_types=[
          pltpu.VMEM_SHARED((HALF + 8, F), jnp.float32),
          pltpu.VMEM((ROWS_PER_TILE, F), jnp.float32),
          pltpu.VMEM((CHUNK, F), jnp.float32),
          pltpu.VMEM((CHUNK, 2 * F), jnp.float32),
          pltpu.VMEM((CHUNK,), jnp.int32),
          pltpu.VMEM((CHUNK,), jnp.int32),
          pltpu.VMEM((CHUNK,), jnp.int32),
          pltpu.VMEM((CHUNK,), jnp.int32),
          pltpu.VMEM((CHUNK,), jnp.int32),
          pltpu.SemaphoreType.DMA,
          pltpu.SemaphoreType.DMA,
      ],
  )(k, qv, skip, src, dst)
  return out.reshape(b * t, NPAD, f)[:, :n, :].reshape(b, t, n, f)


# speedup vs baseline: 7.5300x; 1.2785x over previous
import functools

import jax
import jax.numpy as jnp
from jax import lax
from jax.experimental import pallas as pl
from jax.experimental.pallas import tpu as pltpu
from jax.experimental.pallas import tpu_sc as plsc

N = 10000
NPAD = 10240
F = 128
E = 160000
BT = 2
NSUB = 16
HALF = NPAD // 2
TRASH = HALF
ROWS_PER_TILE = HALF // NSUB
EDGES_PER_TILE = E // NSUB
CHUNK = 80
NCHUNK = EDGES_PER_TILE // CHUNK
EPAD = 10240
LANES = 16


def _linear_body(x_ref, wk_ref, bk_ref, wqv_ref, bqv_ref, ws_ref, bs_ref,
                 k_ref, qv_ref, skip_ref):
  xb = x_ref[...]
  k_ref[...] = (
      jnp.dot(xb, wk_ref[...], preferred_element_type=jnp.float32)
      + bk_ref[...])
  qv_ref[...] = (
      jnp.dot(xb, wqv_ref[...], preferred_element_type=jnp.float32)
      + bqv_ref[...])
  skip_ref[...] = (
      jnp.dot(xb, ws_ref[...], preferred_element_type=jnp.float32)
      + bs_ref[...])


def _tc_linear(x_flat, wk, bk, wqv, bqv, ws, bs):
  rows = x_flat.shape[0]
  blk = 640
  grid = rows // blk
  full = lambda i: (0, 0)
  out = pl.pallas_call(
      _linear_body,
      grid=(grid,),
      in_specs=[
          pl.BlockSpec((blk, F), lambda i: (i, 0)),
          pl.BlockSpec((F, F), full),
          pl.BlockSpec((1, F), full),
          pl.BlockSpec((F, 2 * F), full),
          pl.BlockSpec((1, 2 * F), full),
          pl.BlockSpec((F, F), full),
          pl.BlockSpec((1, F), full),
      ],
      out_specs=[
          pl.BlockSpec((blk, F), lambda i: (i, 0)),
          pl.BlockSpec((blk, 2 * F), lambda i: (i, 0)),
          pl.BlockSpec((blk, F), lambda i: (i, 0)),
      ],
      out_shape=[
          jax.ShapeDtypeStruct((rows, F), jnp.float32),
          jax.ShapeDtypeStruct((rows, 2 * F), jnp.float32),
          jax.ShapeDtypeStruct((rows, F), jnp.float32),
      ],
  )(x_flat, wk, bk, wqv, bqv, ws, bs)
  return out


def _sc_body(k_hbm, qv_hbm, skip_hbm, src_hbm, dst_hbm, out_hbm,
             agg_sh, slab_v, kb0, kb1, qvb0, qvb1,
             sidx0, didx0, sidx1, didx1, scat0, scat1,
             sem_k0, sem_qv0, sem_k1, sem_qv1):
  c = lax.axis_index("c")
  t = lax.axis_index("s")
  trash = jnp.full((LANES,), TRASH, dtype=jnp.int32)
  SLAB = ROWS_PER_TILE // 4

  for b in range(BT):
    row0 = b * NPAD + c * HALF + t * ROWS_PER_TILE

    for h in range(4):
      pltpu.sync_copy(skip_hbm.at[pl.ds(row0 + h * SLAB, SLAB)], slab_v)
      pltpu.sync_copy(
          slab_v, agg_sh.at[pl.ds(t * ROWS_PER_TILE + h * SLAB, SLAB)])
    plsc.subcore_barrier()

    bN = jnp.full((LANES,), b * NPAD, dtype=jnp.int32)
    sub = bN + jnp.full((LANES,), 1, jnp.int32) * c * HALF

    def gather_start(ci, kb, qvb, sidx, didx, semk, semq):
      row = t * NCHUNK + ci
      pltpu.sync_copy(src_hbm.at[row], sidx)
      pltpu.sync_copy(dst_hbm.at[row], didx)
      for i in range(CHUNK // LANES):
        sl = pl.ds(i * LANES, LANES)
        sidx[sl] = sidx[sl] + bN
        didx[sl] = didx[sl] + bN
      pltpu.async_copy(k_hbm.at[didx], kb, semk)
      pltpu.async_copy(qv_hbm.at[sidx], qvb, semq)

    def process(ci, kb, qvb, sidx, didx, scat, semk, semq):
      pltpu.make_async_copy(k_hbm.at[didx], kb, semk).wait()
      pltpu.make_async_copy(qv_hbm.at[sidx], qvb, semq).wait()
      for i in range(CHUNK // LANES):
        ld = didx[pl.ds(i * LANES, LANES)] - sub
        keep = (ld >= 0) & (ld < HALF)
        scat[pl.ds(i * LANES, LANES)] = jnp.where(keep, ld, trash)

      def edge_body(e, carry):
        for j in range(F // LANES):
          sl = pl.ds(j * LANES, LANES)
          eta = kb[e, sl] + qvb[e, sl]
          eta = jnp.maximum(eta, eta * 0.01)
          kb[e, sl] = eta * qvb[e, pl.ds(F + j * LANES, LANES)]
        return carry

      lax.fori_loop(0, CHUNK, edge_body, 0)
      pltpu.sync_copy(kb, agg_sh.at[scat], add=True)

    gather_start(0, kb0, qvb0, sidx0, didx0, sem_k0, sem_qv0)

    def outer(i, carry):
      g = i * 2
      gather_start(g + 1, kb1, qvb1, sidx1, didx1, sem_k1, sem_qv1)
      process(g, kb0, qvb0, sidx0, didx0, scat0, sem_k0, sem_qv0)
      gather_start(g + 2, kb0, qvb0, sidx0, didx0, sem_k0, sem_qv0)
      process(g + 1, kb1, qvb1, sidx1, didx1, scat1, sem_k1, sem_qv1)
      return carry

    lax.fori_loop(0, (NCHUNK - 1) // 2, outer, 0)
    process(NCHUNK - 1, kb0, qvb0, sidx0, didx0, scat0, sem_k0, sem_qv0)
    plsc.subcore_barrier()

    for h in range(4):
      pltpu.sync_copy(
          agg_sh.at[pl.ds(t * ROWS_PER_TILE + h * SLAB, SLAB)], slab_v)
      pltpu.sync_copy(slab_v, out_hbm.at[pl.ds(row0 + h * SLAB, SLAB)])
    plsc.subcore_barrier()


def kernel(x, edge_index, Wk, bk, Wq, bq, Wv, bv, Ws, bias):
  b, t, n, f = x.shape
  x_pad = jnp.pad(x.reshape(b * t, n, f), ((0, 0), (0, NPAD - n), (0, 0)))
  x_flat = x_pad.reshape(b * t * NPAD, f)
  wqv = jnp.concatenate([Wq, Wv], axis=1)
  bqv = jnp.concatenate([bq, bv])[None, :]
  k, qv, skip = _tc_linear(x_flat, Wk, bk[None, :], wqv, bqv, Ws,
                           bias[None, :])

  src = edge_index[0].reshape(E // CHUNK, CHUNK)
  dst = edge_index[1].reshape(E // CHUNK, CHUNK)

  mesh = plsc.VectorSubcoreMesh(core_axis_name="c", subcore_axis_name="s")
  out = pl.kernel(
      _sc_body,
      out_type=jax.ShapeDtypeStruct((b * t * NPAD, f), jnp.float32),
      mesh=mesh,
      compiler_params=pltpu.CompilerParams(use_tc_tiling_on_sc=False),
      scratch_types=[
          pltpu.VMEM_SHARED((HALF + 8, F), jnp.float32),
          pltpu.VMEM((ROWS_PER_TILE // 4, F), jnp.float32),
          pltpu.VMEM((CHUNK, F), jnp.float32),
          pltpu.VMEM((CHUNK, F), jnp.float32),
          pltpu.VMEM((CHUNK, 2 * F), jnp.float32),
          pltpu.VMEM((CHUNK, 2 * F), jnp.float32),
          pltpu.VMEM((CHUNK,), jnp.int32),
          pltpu.VMEM((CHUNK,), jnp.int32),
          pltpu.VMEM((CHUNK,), jnp.int32),
          pltpu.VMEM((CHUNK,), jnp.int32),
          pltpu.VMEM((CHUNK,), jnp.int32),
          pltpu.VMEM((CHUNK,), jnp.int32),
          pltpu.SemaphoreType.DMA,
          pltpu.SemaphoreType.DMA,
          pltpu.SemaphoreType.DMA,
          pltpu.SemaphoreType.DMA,
      ],
  )(k, qv, skip, src, dst)
  return out.reshape(b * t, NPAD, f)[:, :n, :].reshape(b, t, n, f)


# speedup vs baseline: 7.6058x; 1.0101x over previous
import functools

import jax
import jax.numpy as jnp
from jax import lax
from jax.experimental import pallas as pl
from jax.experimental.pallas import tpu as pltpu
from jax.experimental.pallas import tpu_sc as plsc

N = 10000
NPAD = 10240
F = 128
E = 160000
BT = 2
NSUB = 16
HALF = NPAD // 2
TRASH = HALF
ROWS_PER_TILE = HALF // NSUB
EDGES_PER_TILE = E // NSUB
CHUNK = 80
NCHUNK = EDGES_PER_TILE // CHUNK
EPAD = 10240
LANES = 16


def _linear_body(x_ref, wk_ref, bk_ref, wqv_ref, bqv_ref, ws_ref, bs_ref,
                 k_ref, qv_ref, skip_ref):
  xb = x_ref[...]
  k_ref[...] = (
      jnp.dot(xb, wk_ref[...], preferred_element_type=jnp.float32)
      + bk_ref[...])
  qv_ref[...] = (
      jnp.dot(xb, wqv_ref[...], preferred_element_type=jnp.float32)
      + bqv_ref[...])
  skip_ref[...] = (
      jnp.dot(xb, ws_ref[...], preferred_element_type=jnp.float32)
      + bs_ref[...])


def _tc_linear(x_flat, wk, bk, wqv, bqv, ws, bs):
  rows = x_flat.shape[0]
  blk = 640
  grid = rows // blk
  full = lambda i: (0, 0)
  out = pl.pallas_call(
      _linear_body,
      grid=(grid,),
      in_specs=[
          pl.BlockSpec((blk, F), lambda i: (i, 0)),
          pl.BlockSpec((F, F), full),
          pl.BlockSpec((1, F), full),
          pl.BlockSpec((F, 2 * F), full),
          pl.BlockSpec((1, 2 * F), full),
          pl.BlockSpec((F, F), full),
          pl.BlockSpec((1, F), full),
      ],
      out_specs=[
          pl.BlockSpec((blk, F), lambda i: (i, 0)),
          pl.BlockSpec((blk, 2 * F), lambda i: (i, 0)),
          pl.BlockSpec((blk, F), lambda i: (i, 0)),
      ],
      out_shape=[
          jax.ShapeDtypeStruct((rows, F), jnp.float32),
          jax.ShapeDtypeStruct((rows, 2 * F), jnp.float32),
          jax.ShapeDtypeStruct((rows, F), jnp.float32),
      ],
  )(x_flat, wk, bk, wqv, bqv, ws, bs)
  return out


def _sc_body(k_hbm, qv_hbm, skip_hbm, src_hbm, dst_hbm, out_hbm,
             agg_sh, slab_v, kb0, kb1, qvb0, qvb1,
             sidx0, didx0, sidx1, didx1, scat0, scat1,
             sem_k0, sem_qv0, sem_k1, sem_qv1):
  c = lax.axis_index("c")
  t = lax.axis_index("s")
  trash = jnp.full((LANES,), TRASH, dtype=jnp.int32)
  SLAB = ROWS_PER_TILE // 4

  for b in range(BT):
    row0 = b * NPAD + c * HALF + t * ROWS_PER_TILE

    for h in range(4):
      pltpu.sync_copy(skip_hbm.at[pl.ds(row0 + h * SLAB, SLAB)], slab_v)
      pltpu.sync_copy(
          slab_v, agg_sh.at[pl.ds(t * ROWS_PER_TILE + h * SLAB, SLAB)])
    plsc.subcore_barrier()

    bN = jnp.full((LANES,), b * NPAD, dtype=jnp.int32)
    sub = bN + jnp.full((LANES,), 1, jnp.int32) * c * HALF

    def gather_start(ci, kb, qvb, sidx, didx, semk, semq):
      row = t * NCHUNK + ci
      pltpu.sync_copy(src_hbm.at[row], sidx)
      pltpu.sync_copy(dst_hbm.at[row], didx)
      for i in range(CHUNK // LANES):
        sl = pl.ds(i * LANES, LANES)
        sidx[sl] = sidx[sl] + bN
        didx[sl] = didx[sl] + bN
      pltpu.async_copy(k_hbm.at[didx], kb, semk)
      pltpu.async_copy(qv_hbm.at[sidx], qvb, semq)

    def process(ci, kb, qvb, sidx, didx, scat, semk, semq):
      pltpu.make_async_copy(k_hbm.at[didx], kb, semk).wait()
      pltpu.make_async_copy(qv_hbm.at[sidx], qvb, semq).wait()
      for i in range(CHUNK // LANES):
        ld = didx[pl.ds(i * LANES, LANES)] - sub
        keep = (ld >= 0) & (ld < HALF)
        scat[pl.ds(i * LANES, LANES)] = jnp.where(keep, ld, trash)

      def edge_body(e, carry):
        for j in range(F // LANES):
          sl = pl.ds(j * LANES, LANES)
          eta = kb[e, sl] + qvb[e, sl]
          eta = jnp.maximum(eta, eta * 0.01)
          kb[e, sl] = eta * qvb[e, pl.ds(F + j * LANES, LANES)]
        return carry

      lax.fori_loop(0, CHUNK, edge_body, 0, unroll=4)
      pltpu.sync_copy(kb, agg_sh.at[scat], add=True)

    gather_start(0, kb0, qvb0, sidx0, didx0, sem_k0, sem_qv0)

    def outer(i, carry):
      g = i * 2
      gather_start(g + 1, kb1, qvb1, sidx1, didx1, sem_k1, sem_qv1)
      process(g, kb0, qvb0, sidx0, didx0, scat0, sem_k0, sem_qv0)
      gather_start(g + 2, kb0, qvb0, sidx0, didx0, sem_k0, sem_qv0)
      process(g + 1, kb1, qvb1, sidx1, didx1, scat1, sem_k1, sem_qv1)
      return carry

    lax.fori_loop(0, (NCHUNK - 1) // 2, outer, 0)
    process(NCHUNK - 1, kb0, qvb0, sidx0, didx0, scat0, sem_k0, sem_qv0)
    plsc.subcore_barrier()

    for h in range(4):
      pltpu.sync_copy(
          agg_sh.at[pl.ds(t * ROWS_PER_TILE + h * SLAB, SLAB)], slab_v)
      pltpu.sync_copy(slab_v, out_hbm.at[pl.ds(row0 + h * SLAB, SLAB)])
    plsc.subcore_barrier()


def kernel(x, edge_index, Wk, bk, Wq, bq, Wv, bv, Ws, bias):
  b, t, n, f = x.shape
  x_pad = jnp.pad(x.reshape(b * t, n, f), ((0, 0), (0, NPAD - n), (0, 0)))
  x_flat = x_pad.reshape(b * t * NPAD, f)
  wqv = jnp.concatenate([Wq, Wv], axis=1)
  bqv = jnp.concatenate([bq, bv])[None, :]
  k, qv, skip = _tc_linear(x_flat, Wk, bk[None, :], wqv, bqv, Ws,
                           bias[None, :])

  src = edge_index[0].reshape(E // CHUNK, CHUNK)
  dst = edge_index[1].reshape(E // CHUNK, CHUNK)

  mesh = plsc.VectorSubcoreMesh(core_axis_name="c", subcore_axis_name="s")
  out = pl.kernel(
      _sc_body,
      out_type=jax.ShapeDtypeStruct((b * t * NPAD, f), jnp.float32),
      mesh=mesh,
      compiler_params=pltpu.CompilerParams(use_tc_tiling_on_sc=False),
      scratch_types=[
          pltpu.VMEM_SHARED((HALF + 8, F), jnp.float32),
          pltpu.VMEM((ROWS_PER_TILE // 4, F), jnp.float32),
          pltpu.VMEM((CHUNK, F), jnp.float32),
          pltpu.VMEM((CHUNK, F), jnp.float32),
          pltpu.VMEM((CHUNK, 2 * F), jnp.float32),
          pltpu.VMEM((CHUNK, 2 * F), jnp.float32),
          pltpu.VMEM((CHUNK,), jnp.int32),
          pltpu.VMEM((CHUNK,), jnp.int32),
          pltpu.VMEM((CHUNK,), jnp.int32),
          pltpu.VMEM((CHUNK,), jnp.int32),
          pltpu.VMEM((CHUNK,), jnp.int32),
          pltpu.VMEM((CHUNK,), jnp.int32),
          pltpu.SemaphoreType.DMA,
          pltpu.SemaphoreType.DMA,
          pltpu.SemaphoreType.DMA,
          pltpu.SemaphoreType.DMA,
      ],
  )(k, qv, skip, src, dst)
  return out.reshape(b * t, NPAD, f)[:, :n, :].reshape(b, t, n, f)


# speedup vs baseline: 16.3772x; 2.1532x over previous
import functools

import jax
import jax.numpy as jnp
from jax import lax
from jax.experimental import pallas as pl
from jax.experimental.pallas import tpu as pltpu
from jax.experimental.pallas import tpu_sc as plsc

N = 10000
NPAD = 10240
F = 128
E = 160000
BT = 2
NSUB = 16
HALF = NPAD // 2
TRASH = HALF
ROWS_PER_TILE = HALF // NSUB
EDGES_PER_TILE = E // NSUB
CHUNK = 80
NCHUNK = EDGES_PER_TILE // CHUNK
EPAD = 10240
LANES = 16


def _linear_body(x_ref, wk_ref, bk_ref, wqv_ref, bqv_ref, ws_ref, bs_ref,
                 k_ref, qv_ref, skip_ref):
  xb = x_ref[...]
  k_ref[...] = (
      jnp.dot(xb, wk_ref[...], preferred_element_type=jnp.float32)
      + bk_ref[...])
  qv_ref[...] = (
      jnp.dot(xb, wqv_ref[...], preferred_element_type=jnp.float32)
      + bqv_ref[...])
  skip_ref[...] = (
      jnp.dot(xb, ws_ref[...], preferred_element_type=jnp.float32)
      + bs_ref[...])


def _tc_linear(x_flat, wk, bk, wqv, bqv, ws, bs):
  rows = x_flat.shape[0]
  blk = 640
  grid = rows // blk
  full = lambda i: (0, 0)
  out = pl.pallas_call(
      _linear_body,
      grid=(grid,),
      in_specs=[
          pl.BlockSpec((blk, F), lambda i: (i, 0)),
          pl.BlockSpec((F, F), full),
          pl.BlockSpec((1, F), full),
          pl.BlockSpec((F, 2 * F), full),
          pl.BlockSpec((1, 2 * F), full),
          pl.BlockSpec((F, F), full),
          pl.BlockSpec((1, F), full),
      ],
      out_specs=[
          pl.BlockSpec((blk, F), lambda i: (i, 0)),
          pl.BlockSpec((blk, 2 * F), lambda i: (i, 0)),
          pl.BlockSpec((blk, F), lambda i: (i, 0)),
      ],
      out_shape=[
          jax.ShapeDtypeStruct((rows, F), jnp.float32),
          jax.ShapeDtypeStruct((rows, 2 * F), jnp.float32),
          jax.ShapeDtypeStruct((rows, F), jnp.float32),
      ],
  )(x_flat, wk, bk, wqv, bqv, ws, bs)
  return out


def _sc_body(k_hbm, qv_hbm, skip_hbm, src_hbm, dst_hbm, out_hbm,
             agg_sh, slab_v, kb0, kb1, qvb0, qvb1,
             sidx0, didx0, sidx1, didx1, scat0, scat1,
             sem_k0, sem_qv0, sem_k1, sem_qv1):
  c = lax.axis_index("c")
  t = lax.axis_index("s")
  trash = jnp.full((LANES,), TRASH, dtype=jnp.int32)
  SLAB = ROWS_PER_TILE // 4

  for b in range(BT):
    row0 = b * NPAD + c * HALF + t * ROWS_PER_TILE

    for h in range(4):
      pltpu.sync_copy(skip_hbm.at[pl.ds(row0 + h * SLAB, SLAB)], slab_v)
      pltpu.sync_copy(
          slab_v, agg_sh.at[pl.ds(t * ROWS_PER_TILE + h * SLAB, SLAB)])
    plsc.subcore_barrier()

    bN = jnp.full((LANES,), b * NPAD, dtype=jnp.int32)
    sub = bN + jnp.full((LANES,), 1, jnp.int32) * c * HALF

    def gather_start(ci, kb, qvb, sidx, didx, semk, semq):
      row = t * NCHUNK + ci
      pltpu.sync_copy(src_hbm.at[row], sidx)
      pltpu.sync_copy(dst_hbm.at[row], didx)
      for i in range(CHUNK // LANES):
        sl = pl.ds(i * LANES, LANES)
        sidx[sl] = sidx[sl] + bN
        didx[sl] = didx[sl] + bN
      pltpu.async_copy(k_hbm.at[didx], kb, semk)
      pltpu.async_copy(qv_hbm.at[sidx], qvb, semq)

    def process(ci, kb, qvb, sidx, didx, scat, semk, semq):
      pltpu.make_async_copy(k_hbm.at[didx], kb, semk).wait()
      pltpu.make_async_copy(qv_hbm.at[sidx], qvb, semq).wait()
      for i in range(CHUNK // LANES):
        ld = didx[pl.ds(i * LANES, LANES)] - sub
        keep = (ld >= 0) & (ld < HALF)
        scat[pl.ds(i * LANES, LANES)] = jnp.where(keep, ld, trash)

      @plsc.parallel_loop(0, CHUNK, unroll=2)
      def edge_body(e):
        for j in range(F // LANES):
          sl = pl.ds(j * LANES, LANES)
          eta = kb[e, sl] + qvb[e, sl]
          eta = jnp.maximum(eta, eta * 0.01)
          kb[e, sl] = eta * qvb[e, pl.ds(F + j * LANES, LANES)]
      pltpu.sync_copy(kb, agg_sh.at[scat], add=True)

    gather_start(0, kb0, qvb0, sidx0, didx0, sem_k0, sem_qv0)

    def outer(i, carry):
      g = i * 2
      gather_start(g + 1, kb1, qvb1, sidx1, didx1, sem_k1, sem_qv1)
      process(g, kb0, qvb0, sidx0, didx0, scat0, sem_k0, sem_qv0)
      gather_start(g + 2, kb0, qvb0, sidx0, didx0, sem_k0, sem_qv0)
      process(g + 1, kb1, qvb1, sidx1, didx1, scat1, sem_k1, sem_qv1)
      return carry

    lax.fori_loop(0, (NCHUNK - 1) // 2, outer, 0)
    process(NCHUNK - 1, kb0, qvb0, sidx0, didx0, scat0, sem_k0, sem_qv0)
    plsc.subcore_barrier()

    for h in range(4):
      pltpu.sync_copy(
          agg_sh.at[pl.ds(t * ROWS_PER_TILE + h * SLAB, SLAB)], slab_v)
      pltpu.sync_copy(slab_v, out_hbm.at[pl.ds(row0 + h * SLAB, SLAB)])
    plsc.subcore_barrier()


def kernel(x, edge_index, Wk, bk, Wq, bq, Wv, bv, Ws, bias):
  b, t, n, f = x.shape
  x_pad = jnp.pad(x.reshape(b * t, n, f), ((0, 0), (0, NPAD - n), (0, 0)))
  x_flat = x_pad.reshape(b * t * NPAD, f)
  wqv = jnp.concatenate([Wq, Wv], axis=1)
  bqv = jnp.concatenate([bq, bv])[None, :]
  k, qv, skip = _tc_linear(x_flat, Wk, bk[None, :], wqv, bqv, Ws,
                           bias[None, :])

  src = edge_index[0].reshape(E // CHUNK, CHUNK)
  dst = edge_index[1].reshape(E // CHUNK, CHUNK)

  mesh = plsc.VectorSubcoreMesh(core_axis_name="c", subcore_axis_name="s")
  out = pl.kernel(
      _sc_body,
      out_type=jax.ShapeDtypeStruct((b * t * NPAD, f), jnp.float32),
      mesh=mesh,
      compiler_params=pltpu.CompilerParams(use_tc_tiling_on_sc=False),
      scratch_types=[
          pltpu.VMEM_SHARED((HALF + 8, F), jnp.float32),
          pltpu.VMEM((ROWS_PER_TILE // 4, F), jnp.float32),
          pltpu.VMEM((CHUNK, F), jnp.float32),
          pltpu.VMEM((CHUNK, F), jnp.float32),
          pltpu.VMEM((CHUNK, 2 * F), jnp.float32),
          pltpu.VMEM((CHUNK, 2 * F), jnp.float32),
          pltpu.VMEM((CHUNK,), jnp.int32),
          pltpu.VMEM((CHUNK,), jnp.int32),
          pltpu.VMEM((CHUNK,), jnp.int32),
          pltpu.VMEM((CHUNK,), jnp.int32),
          pltpu.VMEM((CHUNK,), jnp.int32),
          pltpu.VMEM((CHUNK,), jnp.int32),
          pltpu.SemaphoreType.DMA,
          pltpu.SemaphoreType.DMA,
          pltpu.SemaphoreType.DMA,
          pltpu.SemaphoreType.DMA,
      ],
  )(k, qv, skip, src, dst)
  return out.reshape(b * t, NPAD, f)[:, :n, :].reshape(b, t, n, f)


# speedup vs baseline: 18.2055x; 1.1116x over previous
import functools

import jax
import jax.numpy as jnp
from jax import lax
from jax.experimental import pallas as pl
from jax.experimental.pallas import tpu as pltpu
from jax.experimental.pallas import tpu_sc as plsc

N = 10000
NPAD = 10240
F = 128
E = 160000
BT = 2
NSUB = 16
HALF = NPAD // 2
TRASH = HALF
ROWS_PER_TILE = HALF // NSUB
EDGES_PER_TILE = E // NSUB
CHUNK = 80
NCHUNK = EDGES_PER_TILE // CHUNK
EPAD = 10240
LANES = 16


def _linear_body(x_ref, wk_ref, bk_ref, wqv_ref, bqv_ref, ws_ref, bs_ref,
                 k_ref, qv_ref, skip_ref):
  xb = x_ref[...]
  k_ref[...] = (
      jnp.dot(xb, wk_ref[...], preferred_element_type=jnp.float32)
      + bk_ref[...])
  qv_ref[...] = (
      jnp.dot(xb, wqv_ref[...], preferred_element_type=jnp.float32)
      + bqv_ref[...])
  skip_ref[...] = (
      jnp.dot(xb, ws_ref[...], preferred_element_type=jnp.float32)
      + bs_ref[...])


def _tc_linear(x_flat, wk, bk, wqv, bqv, ws, bs):
  rows = x_flat.shape[0]
  blk = 640
  grid = rows // blk
  full = lambda i: (0, 0)
  out = pl.pallas_call(
      _linear_body,
      grid=(grid,),
      in_specs=[
          pl.BlockSpec((blk, F), lambda i: (i, 0)),
          pl.BlockSpec((F, F), full),
          pl.BlockSpec((1, F), full),
          pl.BlockSpec((F, 2 * F), full),
          pl.BlockSpec((1, 2 * F), full),
          pl.BlockSpec((F, F), full),
          pl.BlockSpec((1, F), full),
      ],
      out_specs=[
          pl.BlockSpec((blk, F), lambda i: (i, 0)),
          pl.BlockSpec((blk, 2 * F), lambda i: (i, 0)),
          pl.BlockSpec((blk, F), lambda i: (i, 0)),
      ],
      out_shape=[
          jax.ShapeDtypeStruct((rows, F), jnp.float32),
          jax.ShapeDtypeStruct((rows, 2 * F), jnp.float32),
          jax.ShapeDtypeStruct((rows, F), jnp.float32),
      ],
  )(x_flat, wk, bk, wqv, bqv, ws, bs)
  return out


def _sc_body(k_hbm, qv_hbm, skip_hbm, sd_hbm, out_hbm,
             agg_sh, slab_v, kb0, kb1, qvb0, qvb1,
             sd0, sd1, scat0, scat1,
             sem_k0, sem_qv0, sem_k1, sem_qv1):
  c = lax.axis_index("c")
  t = lax.axis_index("s")
  trash = jnp.full((LANES,), TRASH, dtype=jnp.int32)
  SLAB = ROWS_PER_TILE // 4

  for b in range(BT):
    row0 = b * NPAD + c * HALF + t * ROWS_PER_TILE

    for h in range(4):
      pltpu.sync_copy(skip_hbm.at[pl.ds(row0 + h * SLAB, SLAB)], slab_v)
      pltpu.sync_copy(
          slab_v, agg_sh.at[pl.ds(t * ROWS_PER_TILE + h * SLAB, SLAB)])
    plsc.subcore_barrier()

    bN = jnp.full((LANES,), b * NPAD, dtype=jnp.int32)
    sub = bN + jnp.full((LANES,), 1, jnp.int32) * c * HALF

    def gather_start(ci, kb, qvb, sd, semk, semq):
      row = t * NCHUNK + ci
      pltpu.sync_copy(sd_hbm.at[row], sd)
      for i in range(CHUNK // LANES):
        sl = pl.ds(i * LANES, LANES)
        sd[0, sl] = sd[0, sl] + bN
        sd[1, sl] = sd[1, sl] + bN
      pltpu.async_copy(k_hbm.at[sd.at[1]], kb, semk)
      pltpu.async_copy(qv_hbm.at[sd.at[0]], qvb, semq)

    def process(ci, kb, qvb, sd, scat, semk, semq):
      pltpu.make_async_copy(k_hbm.at[sd.at[1]], kb, semk).wait()
      pltpu.make_async_copy(qv_hbm.at[sd.at[0]], qvb, semq).wait()
      for i in range(CHUNK // LANES):
        ld = sd[1, pl.ds(i * LANES, LANES)] - sub
        keep = (ld >= 0) & (ld < HALF)
        scat[pl.ds(i * LANES, LANES)] = jnp.where(keep, ld, trash)

      @plsc.parallel_loop(0, CHUNK, unroll=4)
      def edge_body(e):
        for j in range(F // LANES):
          sl = pl.ds(j * LANES, LANES)
          eta = kb[e, sl] + qvb[e, sl]
          eta = jnp.maximum(eta, eta * 0.01)
          kb[e, sl] = eta * qvb[e, pl.ds(F + j * LANES, LANES)]
      pltpu.sync_copy(kb, agg_sh.at[scat], add=True)

    gather_start(0, kb0, qvb0, sd0, sem_k0, sem_qv0)

    def outer(i, carry):
      g = i * 2
      gather_start(g + 1, kb1, qvb1, sd1, sem_k1, sem_qv1)
      process(g, kb0, qvb0, sd0, scat0, sem_k0, sem_qv0)
      gather_start(g + 2, kb0, qvb0, sd0, sem_k0, sem_qv0)
      process(g + 1, kb1, qvb1, sd1, scat1, sem_k1, sem_qv1)
      return carry

    lax.fori_loop(0, (NCHUNK - 1) // 2, outer, 0)
    process(NCHUNK - 1, kb0, qvb0, sd0, scat0, sem_k0, sem_qv0)
    plsc.subcore_barrier()

    for h in range(4):
      pltpu.sync_copy(
          agg_sh.at[pl.ds(t * ROWS_PER_TILE + h * SLAB, SLAB)], slab_v)
      pltpu.sync_copy(slab_v, out_hbm.at[pl.ds(row0 + h * SLAB, SLAB)])
    plsc.subcore_barrier()


def kernel(x, edge_index, Wk, bk, Wq, bq, Wv, bv, Ws, bias):
  b, t, n, f = x.shape
  x_pad = jnp.pad(x.reshape(b * t, n, f), ((0, 0), (0, NPAD - n), (0, 0)))
  x_flat = x_pad.reshape(b * t * NPAD, f)
  wqv = jnp.concatenate([Wq, Wv], axis=1)
  bqv = jnp.concatenate([bq, bv])[None, :]
  k, qv, skip = _tc_linear(x_flat, Wk, bk[None, :], wqv, bqv, Ws,
                           bias[None, :])

  sd = jnp.stack([edge_index[0].reshape(E // CHUNK, CHUNK),
                  edge_index[1].reshape(E // CHUNK, CHUNK)], axis=1)

  mesh = plsc.VectorSubcoreMesh(core_axis_name="c", subcore_axis_name="s")
  out = pl.kernel(
      _sc_body,
      out_type=jax.ShapeDtypeStruct((b * t * NPAD, f), jnp.float32),
      mesh=mesh,
      compiler_params=pltpu.CompilerParams(use_tc_tiling_on_sc=False),
      scratch_types=[
          pltpu.VMEM_SHARED((HALF + 8, F), jnp.float32),
          pltpu.VMEM((ROWS_PER_TILE // 4, F), jnp.float32),
          pltpu.VMEM((CHUNK, F), jnp.float32),
          pltpu.VMEM((CHUNK, F), jnp.float32),
          pltpu.VMEM((CHUNK, 2 * F), jnp.float32),
          pltpu.VMEM((CHUNK, 2 * F), jnp.float32),
          pltpu.VMEM((2, CHUNK), jnp.int32),
          pltpu.VMEM((2, CHUNK), jnp.int32),
          pltpu.VMEM((CHUNK,), jnp.int32),
          pltpu.VMEM((CHUNK,), jnp.int32),
          pltpu.SemaphoreType.DMA,
          pltpu.SemaphoreType.DMA,
          pltpu.SemaphoreType.DMA,
          pltpu.SemaphoreType.DMA,
      ],
  )(k, qv, skip, sd)
  return out.reshape(b * t, NPAD, f)[:, :n, :].reshape(b, t, n, f)


# speedup vs baseline: 18.8416x; 1.0349x over previous
import functools

import jax
import jax.numpy as jnp
from jax import lax
from jax.experimental import pallas as pl
from jax.experimental.pallas import tpu as pltpu
from jax.experimental.pallas import tpu_sc as plsc

N = 10000
NPAD = 10240
F = 128
E = 160000
BT = 2
NSUB = 16
HALF = NPAD // 2
TRASH = HALF
ROWS_PER_TILE = HALF // NSUB
EDGES_PER_TILE = E // NSUB
CHUNK = 80
NCHUNK = EDGES_PER_TILE // CHUNK
EPAD = 10240
LANES = 16


def _linear_body(x_ref, wk_ref, bk_ref, wqv_ref, bqv_ref, ws_ref, bs_ref,
                 k_ref, qv_ref, skip_ref):
  xb = x_ref[...]
  k_ref[...] = (
      jnp.dot(xb, wk_ref[...], preferred_element_type=jnp.float32)
      + bk_ref[...])
  qv_ref[...] = (
      jnp.dot(xb, wqv_ref[...], preferred_element_type=jnp.float32)
      + bqv_ref[...])
  skip_ref[...] = (
      jnp.dot(xb, ws_ref[...], preferred_element_type=jnp.float32)
      + bs_ref[...])


def _tc_linear(x_flat, wk, bk, wqv, bqv, ws, bs):
  rows = x_flat.shape[0]
  blk = 640
  grid = rows // blk
  full = lambda i: (0, 0)
  out = pl.pallas_call(
      _linear_body,
      grid=(grid,),
      in_specs=[
          pl.BlockSpec((blk, F), lambda i: (i, 0)),
          pl.BlockSpec((F, F), full),
          pl.BlockSpec((1, F), full),
          pl.BlockSpec((F, 2 * F), full),
          pl.BlockSpec((1, 2 * F), full),
          pl.BlockSpec((F, F), full),
          pl.BlockSpec((1, F), full),
      ],
      out_specs=[
          pl.BlockSpec((blk, F), lambda i: (i, 0)),
          pl.BlockSpec((blk, 2 * F), lambda i: (i, 0)),
          pl.BlockSpec((blk, F), lambda i: (i, 0)),
      ],
      out_shape=[
          jax.ShapeDtypeStruct((rows, F), jnp.float32),
          jax.ShapeDtypeStruct((rows, 2 * F), jnp.float32),
          jax.ShapeDtypeStruct((rows, F), jnp.float32),
      ],
  )(x_flat, wk, bk, wqv, bqv, ws, bs)
  return out


def _sc_body(k_hbm, qv_hbm, skip_hbm, sd_hbm, out_hbm,
             agg_sh, kb0, kb1, kb2, qvb0, qvb1,
             sd0, sd1, sd2, scat0, scat1, scat2,
             si0, si1, si2, sk0, sk1, sk2, sq0, sq1, ssc0, ssc1, ssc2):
  c = lax.axis_index("c")
  t = lax.axis_index("s")
  trash = jnp.full((LANES,), TRASH, dtype=jnp.int32)
  SLAB = ROWS_PER_TILE // 4
  kbs = (kb0, kb1, kb2)
  qvbs = (qvb0, qvb1)
  sds = (sd0, sd1, sd2)
  scats = (scat0, scat1, scat2)
  sis = (si0, si1, si2)
  sks = (sk0, sk1, sk2)
  sqs = (sq0, sq1)
  sscs = (ssc0, ssc1, ssc2)

  for b in range(BT):
    row0 = b * NPAD + c * HALF + t * ROWS_PER_TILE

    for h in range(4):
      pltpu.sync_copy(skip_hbm.at[pl.ds(row0 + h * SLAB, SLAB)], kb0)
      pltpu.sync_copy(
          kb0, agg_sh.at[pl.ds(t * ROWS_PER_TILE + h * SLAB, SLAB)])
    plsc.subcore_barrier()

    bN = jnp.full((LANES,), b * NPAD, dtype=jnp.int32)
    sub = bN + jnp.full((LANES,), 1, jnp.int32) * c * HALF

    def idx_start_s(ci, s3):
      pltpu.async_copy(sd_hbm.at[t * NCHUNK + ci], sds[s3], sis[s3])

    def gath_k_s(ci, s3):
      sd = sds[s3]
      pltpu.make_async_copy(sd_hbm.at[t * NCHUNK + ci], sd, sis[s3]).wait()
      for i in range(CHUNK // LANES):
        sl = pl.ds(i * LANES, LANES)
        sd[0, sl] = sd[0, sl] + bN
        sd[1, sl] = sd[1, sl] + bN
      pltpu.async_copy(k_hbm.at[sd.at[1]], kbs[s3], sks[s3])

    def gath_qv_s(ci, s3, s2):
      pltpu.async_copy(qv_hbm.at[sds[s3].at[0]], qvbs[s2], sqs[s2])

    def sc_wait_s(s3):
      pltpu.make_async_copy(kbs[s3], agg_sh.at[scats[s3]], sscs[s3]).wait()

    def proc_s(ci, s3, s2):
      kb = kbs[s3]
      qvb = qvbs[s2]
      sd = sds[s3]
      scat = scats[s3]
      pltpu.make_async_copy(k_hbm.at[sd.at[1]], kb, sks[s3]).wait()
      pltpu.make_async_copy(qv_hbm.at[sd.at[0]], qvb, sqs[s2]).wait()
      for i in range(CHUNK // LANES):
        ld = sd[1, pl.ds(i * LANES, LANES)] - sub
        keep = (ld >= 0) & (ld < HALF)
        scat[pl.ds(i * LANES, LANES)] = jnp.where(keep, ld, trash)

      @plsc.parallel_loop(0, CHUNK, unroll=4)
      def edge_body(e):
        for j in range(F // LANES):
          sl = pl.ds(j * LANES, LANES)
          eta = kb[e, sl] + qvb[e, sl]
          eta = jnp.maximum(eta, eta * 0.01)
          kb[e, sl] = eta * qvb[e, pl.ds(F + j * LANES, LANES)]
      pltpu.async_copy(kb, agg_sh.at[scat], sscs[s3], add=True)

    def body(ci, r, with_wait=True, with_idx=True):
      if with_wait:
        sc_wait_s((r + 2) % 3)
      gath_k_s(ci + 2, (r + 2) % 3)
      gath_qv_s(ci + 1, (r + 1) % 3, (r + 1) % 2)
      proc_s(ci, r % 3, r % 2)
      if with_idx:
        idx_start_s(ci + 3, r % 3)

    idx_start_s(0, 0)
    idx_start_s(1, 1)
    idx_start_s(2, 2)
    gath_k_s(0, 0)
    gath_k_s(1, 1)
    gath_qv_s(0, 0, 0)
    body(0, 0, with_wait=False)

    def outer(i, carry):
      ci0 = 6 * i + 1
      for r in range(6):
        body(ci0 + r, 1 + r)
      return carry

    lax.fori_loop(0, (NCHUNK - 5) // 6, outer, 0)
    body(121, 1)
    body(122, 2, with_idx=False)
    gath_qv_s(124, 1, 0)
    proc_s(123, 0, 1)
    proc_s(124, 1, 0)
    sc_wait_s(2)
    sc_wait_s(0)
    sc_wait_s(1)
    plsc.subcore_barrier()

    for h in range(4):
      pltpu.sync_copy(
          agg_sh.at[pl.ds(t * ROWS_PER_TILE + h * SLAB, SLAB)], kb0)
      pltpu.sync_copy(kb0, out_hbm.at[pl.ds(row0 + h * SLAB, SLAB)])
    plsc.subcore_barrier()


def kernel(x, edge_index, Wk, bk, Wq, bq, Wv, bv, Ws, bias):
  b, t, n, f = x.shape
  x_pad = jnp.pad(x.reshape(b * t, n, f), ((0, 0), (0, NPAD - n), (0, 0)))
  x_flat = x_pad.reshape(b * t * NPAD, f)
  wqv = jnp.concatenate([Wq, Wv], axis=1)
  bqv = jnp.concatenate([bq, bv])[None, :]
  k, qv, skip = _tc_linear(x_flat, Wk, bk[None, :], wqv, bqv, Ws,
                           bias[None, :])

  sd = jnp.stack([edge_index[0].reshape(E // CHUNK, CHUNK),
                  edge_index[1].reshape(E // CHUNK, CHUNK)], axis=1)

  mesh = plsc.VectorSubcoreMesh(core_axis_name="c", subcore_axis_name="s")
  out = pl.kernel(
      _sc_body,
      out_type=jax.ShapeDtypeStruct((b * t * NPAD, f), jnp.float32),
      mesh=mesh,
      compiler_params=pltpu.CompilerParams(use_tc_tiling_on_sc=False),
      scratch_types=[
          pltpu.VMEM_SHARED((HALF + 8, F), jnp.float32),
          pltpu.VMEM((CHUNK, F), jnp.float32),
          pltpu.VMEM((CHUNK, F), jnp.float32),
          pltpu.VMEM((CHUNK, F), jnp.float32),
          pltpu.VMEM((CHUNK, 2 * F), jnp.float32),
          pltpu.VMEM((CHUNK, 2 * F), jnp.float32),
          pltpu.VMEM((2, CHUNK), jnp.int32),
          pltpu.VMEM((2, CHUNK), jnp.int32),
          pltpu.VMEM((2, CHUNK), jnp.int32),
          pltpu.VMEM((CHUNK,), jnp.int32),
          pltpu.VMEM((CHUNK,), jnp.int32),
          pltpu.VMEM((CHUNK,), jnp.int32),
          pltpu.SemaphoreType.DMA,
          pltpu.SemaphoreType.DMA,
          pltpu.SemaphoreType.DMA,
          pltpu.SemaphoreType.DMA,
          pltpu.SemaphoreType.DMA,
          pltpu.SemaphoreType.DMA,
          pltpu.SemaphoreType.DMA,
          pltpu.SemaphoreType.DMA,
          pltpu.SemaphoreType.DMA,
          pltpu.SemaphoreType.DMA,
          pltpu.SemaphoreType.DMA,
      ],
  )(k, qv, skip, sd)
  return out.reshape(b * t, NPAD, f)[:, :n, :].reshape(b, t, n, f)
